# baseline (speedup 1.0000x reference)
import functools
import math

import jax
import jax.numpy as jnp
from jax import lax
from jax.experimental import pallas as pl
from jax.experimental.pallas import tpu as pltpu
from jax.experimental.pallas import tpu_sc as plsc

EMBED_DIM = 64
DIM1 = EMBED_DIM // 3
NUM_SAR = 12
NUM_OPT = 2500
NUM_ROWS = NUM_SAR + NUM_OPT

BATCH = 4096
SEQ = 200

NC, NS = 2, 16
NW = NC * NS
BPW = BATCH // NW
CB = 2
ROWS = CB * SEQ
NCHUNK = BPW // CB


def _table_body(t_ref, r_ref, o_ref, out_ref):
    R, C = NUM_ROWS, EMBED_DIM
    r = lax.broadcasted_iota(jnp.int32, (R, C), 0)
    c = lax.broadcasted_iota(jnp.int32, (R, C), 1)
    pos = (r - NUM_SAR).astype(jnp.float32)
    j = (c % 32).astype(jnp.float32)
    omega = jnp.exp(j * (-math.log(10000.0) / 32.0))
    ang = pos * omega
    sincos = jnp.where(c < 32, jnp.sin(ang), jnp.cos(ang))
    s = 11 - r
    sm4 = s % 4
    q = s // 4
    t0 = jnp.broadcast_to(t_ref[0:1, :], (R, C))
    t1 = jnp.broadcast_to(t_ref[1:2, :], (R, C))
    r0 = jnp.broadcast_to(r_ref[0:1, :], (R, C))
    r1 = jnp.broadcast_to(r_ref[1:2, :], (R, C))
    o0 = jnp.broadcast_to(o_ref[0:1, :], (R, C))
    o1 = jnp.broadcast_to(o_ref[1:2, :], (R, C))
    tv = jnp.where(sm4 < 2, t0, t1)
    rv = jnp.where((sm4 == 0) | (sm4 == 3), r0, r1)
    ov = jnp.where(q == 0, 0.5 * (o0 + o1), jnp.where(q == 1, o0, o1))
    sarv = jnp.where(c < DIM1, tv, jnp.where(c < 2 * DIM1, rv, ov))
    out_ref[...] = jnp.where(r < NUM_SAR, sarv, sincos)


def _build_table(embed_transmit, embed_receive, embed_orbit):
    f32 = jnp.float32
    t = jnp.zeros((2, EMBED_DIM), f32).at[:, 0:DIM1].set(embed_transmit)
    r = jnp.zeros((2, EMBED_DIM), f32).at[:, DIM1:2 * DIM1].set(embed_receive)
    o = jnp.zeros((2, EMBED_DIM), f32).at[:, 2 * DIM1:].set(embed_orbit)
    return pl.pallas_call(
        _table_body,
        out_shape=jax.ShapeDtypeStruct((NUM_ROWS, EMBED_DIM), f32),
    )(t, r, o)


_GATHER_SPLITS = [(0, 128), (128, 72), (200, 128), (328, 72)]


@functools.partial(
    pl.kernel,
    out_type=jax.ShapeDtypeStruct((BATCH, SEQ, EMBED_DIM), jnp.float32),
    mesh=plsc.VectorSubcoreMesh(core_axis_name="c", subcore_axis_name="s"),
    scratch_types=[
        pltpu.VMEM((2 * ROWS,), jnp.int32),
        pltpu.VMEM((2, CB, SEQ, EMBED_DIM), jnp.float32),
        pltpu.VMEM_SHARED((NUM_ROWS, EMBED_DIM), jnp.float32),
        pltpu.SemaphoreType.DMA,
        pltpu.SemaphoreType.DMA,
        pltpu.SemaphoreType.DMA,
        pltpu.SemaphoreType.DMA,
        pltpu.SemaphoreType.DMA,
        pltpu.SemaphoreType.DMA,
    ],
    compiler_params=pltpu.CompilerParams(use_tc_tiling_on_sc=False),
)
def _gather(table_hbm, idx_hbm, out_hbm, idx_v, rows_v, table_sh,
            sem_i0, sem_i1, sem_g0, sem_g1, sem_o0, sem_o1):
    wid = lax.axis_index("s") * NC + lax.axis_index("c")
    rbase = wid * BPW * SEQ
    bbase = wid * BPW
    sem_i = (sem_i0, sem_i1)
    sem_g = (sem_g0, sem_g1)
    sem_o = (sem_o0, sem_o1)

    @pl.when(lax.axis_index("s") == 0)
    def _():
        pltpu.sync_copy(table_hbm, table_sh)
    plsc.subcore_barrier()

    pltpu.async_copy(idx_hbm.at[pl.ds(rbase, ROWS)],
                     idx_v.at[pl.ds(0, ROWS)], sem_i[0])

    def body(i, carry):
        for b in range(2):
            c = 2 * i + b
            roff = c * ROWS
            ioff = b * ROWS

            pltpu.make_async_copy(
                idx_hbm.at[pl.ds(rbase + roff, ROWS)],
                idx_v.at[pl.ds(ioff, ROWS)], sem_i[b]
            ).wait()

            @pl.when(c + 1 < NCHUNK)
            def _():
                pltpu.async_copy(
                    idx_hbm.at[pl.ds(rbase + roff + ROWS, ROWS)],
                    idx_v.at[pl.ds((1 - b) * ROWS, ROWS)],
                    sem_i[1 - b],
                )

            for k in range(ROWS // 16):
                sl = pl.ds(ioff + k * 16, 16)
                idx_v[sl] = idx_v[sl] + NUM_SAR

            @pl.when(i > 0)
            def _():
                pltpu.make_async_copy(
                    rows_v.at[b],
                    out_hbm.at[pl.ds(bbase + c * CB, CB)],
                    sem_o[b],
                ).wait()

            copies = [
                pltpu.async_copy(
                    table_sh.at[idx_v.at[pl.ds(ioff + start, n)]],
                    rows_v.at[b, start // SEQ, pl.ds(start % SEQ, n)],
                    sem_g[b],
                )
                for start, n in _GATHER_SPLITS
            ]
            for cp in copies:
                cp.wait()

            pltpu.async_copy(
                rows_v.at[b], out_hbm.at[pl.ds(bbase + c * CB, CB)], sem_o[b]
            )
        return carry

    lax.fori_loop(0, NCHUNK // 2, body, 0)

    pltpu.make_async_copy(
        rows_v.at[0], out_hbm.at[pl.ds(bbase + (NCHUNK - 2) * CB, CB)], sem_o[0]
    ).wait()
    pltpu.make_async_copy(
        rows_v.at[1], out_hbm.at[pl.ds(bbase + (NCHUNK - 1) * CB, CB)], sem_o[1]
    ).wait()


def _xpose_body(in_ref, out_ref):
    for t in range(SEQ // 2):
        xk = in_ref[:, t, :]
        out_ref[2 * t] = jnp.transpose(xk[:, 0:EMBED_DIM])
        out_ref[2 * t + 1] = jnp.transpose(xk[:, EMBED_DIM:])


def _transpose_tc(out_sc):
    t2 = out_sc.reshape(BATCH, SEQ // 2, 2 * EMBED_DIM)
    return pl.pallas_call(
        _xpose_body,
        grid=(BATCH // BPW,),
        in_specs=[pl.BlockSpec((BPW, SEQ // 2, 2 * EMBED_DIM),
                               lambda g: (g, 0, 0))],
        out_specs=pl.BlockSpec((SEQ, EMBED_DIM, BPW), lambda g: (0, 0, g)),
        out_shape=jax.ShapeDtypeStruct((SEQ, EMBED_DIM, BATCH), jnp.float32),
    )(t2)


def kernel(input, embed_transmit, embed_receive, embed_orbit):
    table = _build_table(embed_transmit, embed_receive, embed_orbit)
    idx = input.reshape(-1).astype(jnp.int32)
    out_sc = _gather(table, idx)
    tr = _transpose_tc(out_sc)
    return jnp.transpose(tr, (2, 0, 1))


# speedup vs baseline: 2.8876x; 2.8876x over previous
import functools
import math

import jax
import jax.numpy as jnp
from jax import lax
from jax.experimental import pallas as pl
from jax.experimental.pallas import tpu as pltpu
from jax.experimental.pallas import tpu_sc as plsc

EMBED_DIM = 64
DIM1 = EMBED_DIM // 3
NUM_SAR = 12
NUM_OPT = 2500
NUM_ROWS = NUM_SAR + NUM_OPT

BATCH = 4096
SEQ = 200

NC, NS = 2, 16
NW = NC * NS
BPW = BATCH // NW
CB = 2
ROWS = CB * SEQ
NCHUNK = BPW // CB


def _table_body(t_ref, r_ref, o_ref, out_ref):
    R, C = NUM_ROWS, EMBED_DIM
    r = lax.broadcasted_iota(jnp.int32, (R, C), 0)
    c = lax.broadcasted_iota(jnp.int32, (R, C), 1)
    pos = (r - NUM_SAR).astype(jnp.float32)
    j = (c % 32).astype(jnp.float32)
    omega = jnp.exp(j * (-math.log(10000.0) / 32.0))
    ang = pos * omega
    sincos = jnp.where(c < 32, jnp.sin(ang), jnp.cos(ang))
    s = 11 - r
    sm4 = s % 4
    q = s // 4
    t0 = jnp.broadcast_to(t_ref[0:1, :], (R, C))
    t1 = jnp.broadcast_to(t_ref[1:2, :], (R, C))
    r0 = jnp.broadcast_to(r_ref[0:1, :], (R, C))
    r1 = jnp.broadcast_to(r_ref[1:2, :], (R, C))
    o0 = jnp.broadcast_to(o_ref[0:1, :], (R, C))
    o1 = jnp.broadcast_to(o_ref[1:2, :], (R, C))
    tv = jnp.where(sm4 < 2, t0, t1)
    rv = jnp.where((sm4 == 0) | (sm4 == 3), r0, r1)
    ov = jnp.where(q == 0, 0.5 * (o0 + o1), jnp.where(q == 1, o0, o1))
    sarv = jnp.where(c < DIM1, tv, jnp.where(c < 2 * DIM1, rv, ov))
    out_ref[...] = jnp.where(r < NUM_SAR, sarv, sincos)


def _build_table(embed_transmit, embed_receive, embed_orbit):
    f32 = jnp.float32
    t = jnp.zeros((2, EMBED_DIM), f32).at[:, 0:DIM1].set(embed_transmit)
    r = jnp.zeros((2, EMBED_DIM), f32).at[:, DIM1:2 * DIM1].set(embed_receive)
    o = jnp.zeros((2, EMBED_DIM), f32).at[:, 2 * DIM1:].set(embed_orbit)
    return pl.pallas_call(
        _table_body,
        out_shape=jax.ShapeDtypeStruct((NUM_ROWS, EMBED_DIM), f32),
    )(t, r, o)


_GATHER_SPLITS = [(0, 128), (128, 72), (200, 128), (328, 72)]


@functools.partial(
    pl.kernel,
    out_type=jax.ShapeDtypeStruct((BATCH, SEQ, EMBED_DIM), jnp.float32),
    mesh=plsc.VectorSubcoreMesh(core_axis_name="c", subcore_axis_name="s"),
    scratch_types=[
        pltpu.VMEM((2 * ROWS,), jnp.int32),
        pltpu.VMEM((2, CB, SEQ, EMBED_DIM), jnp.float32),
        pltpu.VMEM_SHARED((NUM_ROWS, EMBED_DIM), jnp.float32),
        pltpu.SemaphoreType.DMA,
        pltpu.SemaphoreType.DMA,
        pltpu.SemaphoreType.DMA,
        pltpu.SemaphoreType.DMA,
        pltpu.SemaphoreType.DMA,
        pltpu.SemaphoreType.DMA,
    ],
    compiler_params=pltpu.CompilerParams(use_tc_tiling_on_sc=False),
)
def _gather(table_hbm, idx_hbm, out_hbm, idx_v, rows_v, table_sh,
            sem_i0, sem_i1, sem_g0, sem_g1, sem_o0, sem_o1):
    wid = lax.axis_index("s") * NC + lax.axis_index("c")
    rbase = wid * BPW * SEQ
    bbase = wid * BPW
    sem_i = (sem_i0, sem_i1)
    sem_g = (sem_g0, sem_g1)
    sem_o = (sem_o0, sem_o1)

    @pl.when(lax.axis_index("s") == 0)
    def _():
        pltpu.sync_copy(table_hbm, table_sh)
    plsc.subcore_barrier()

    pltpu.async_copy(idx_hbm.at[pl.ds(rbase, ROWS)],
                     idx_v.at[pl.ds(0, ROWS)], sem_i[0])

    def body(i, carry):
        for b in range(2):
            c = 2 * i + b
            roff = c * ROWS
            ioff = b * ROWS

            pltpu.make_async_copy(
                idx_hbm.at[pl.ds(rbase + roff, ROWS)],
                idx_v.at[pl.ds(ioff, ROWS)], sem_i[b]
            ).wait()

            @pl.when(c + 1 < NCHUNK)
            def _():
                pltpu.async_copy(
                    idx_hbm.at[pl.ds(rbase + roff + ROWS, ROWS)],
                    idx_v.at[pl.ds((1 - b) * ROWS, ROWS)],
                    sem_i[1 - b],
                )

            for k in range(ROWS // 16):
                sl = pl.ds(ioff + k * 16, 16)
                idx_v[sl] = idx_v[sl] + NUM_SAR

            @pl.when(i > 0)
            def _():
                pltpu.make_async_copy(
                    rows_v.at[b],
                    out_hbm.at[pl.ds(bbase + c * CB, CB)],
                    sem_o[b],
                ).wait()

            copies = [
                pltpu.async_copy(
                    table_sh.at[idx_v.at[pl.ds(ioff + start, n)]],
                    rows_v.at[b, start // SEQ, pl.ds(start % SEQ, n)],
                    sem_g[b],
                )
                for start, n in _GATHER_SPLITS
            ]
            for cp in copies:
                cp.wait()

            pltpu.async_copy(
                rows_v.at[b], out_hbm.at[pl.ds(bbase + c * CB, CB)], sem_o[b]
            )
        return carry

    lax.fori_loop(0, NCHUNK // 2, body, 0)

    pltpu.make_async_copy(
        rows_v.at[0], out_hbm.at[pl.ds(bbase + (NCHUNK - 2) * CB, CB)], sem_o[0]
    ).wait()
    pltpu.make_async_copy(
        rows_v.at[1], out_hbm.at[pl.ds(bbase + (NCHUNK - 1) * CB, CB)], sem_o[1]
    ).wait()


def _xpose_body(in_ref, out_ref):
    in3 = in_ref.reshape(BPW, SEQ // 2, 2 * EMBED_DIM)
    for t in range(SEQ // 2):
        xk = in3[:, t, :]
        out_ref[2 * t] = jnp.transpose(xk[:, 0:EMBED_DIM])
        out_ref[2 * t + 1] = jnp.transpose(xk[:, EMBED_DIM:])


def _transpose_tc(out_sc):
    t2 = out_sc.reshape(BATCH * SEQ // 2, 2 * EMBED_DIM)
    rows_per_blk = BPW * SEQ // 2
    return pl.pallas_call(
        _xpose_body,
        grid=(BATCH // BPW,),
        in_specs=[pl.BlockSpec((rows_per_blk, 2 * EMBED_DIM),
                               lambda g: (g, 0))],
        out_specs=pl.BlockSpec((SEQ, EMBED_DIM, BPW), lambda g: (0, 0, g)),
        out_shape=jax.ShapeDtypeStruct((SEQ, EMBED_DIM, BATCH), jnp.float32),
    )(t2)


def kernel(input, embed_transmit, embed_receive, embed_orbit):
    table = _build_table(embed_transmit, embed_receive, embed_orbit)
    idx = input.reshape(-1).astype(jnp.int32)
    out_sc = _gather(table, idx)
    tr = _transpose_tc(out_sc)
    return jnp.transpose(tr, (2, 0, 1))


# speedup vs baseline: 3.1729x; 1.0988x over previous
import functools
import math

import jax
import jax.numpy as jnp
from jax import lax
from jax.experimental import pallas as pl
from jax.experimental.pallas import tpu as pltpu
from jax.experimental.pallas import tpu_sc as plsc

EMBED_DIM = 64
DIM1 = EMBED_DIM // 3
NUM_SAR = 12
NUM_OPT = 2500
NUM_ROWS = NUM_SAR + NUM_OPT

BATCH = 4096
SEQ = 200

NC, NS = 2, 16
NW = NC * NS
NSPLIT = 2
NBH = BATCH // NSPLIT
BPW = NBH // NW
CB = 2
ROWS = CB * SEQ
NCHUNK = BPW // CB
TCB = 128


def _table_body(t_ref, r_ref, o_ref, out_ref):
    R, C = NUM_ROWS, EMBED_DIM
    r = lax.broadcasted_iota(jnp.int32, (R, C), 0)
    c = lax.broadcasted_iota(jnp.int32, (R, C), 1)
    pos = (r - NUM_SAR).astype(jnp.float32)
    j = (c % 32).astype(jnp.float32)
    omega = jnp.exp(j * (-math.log(10000.0) / 32.0))
    ang = pos * omega
    sincos = jnp.where(c < 32, jnp.sin(ang), jnp.cos(ang))
    s = 11 - r
    sm4 = s % 4
    q = s // 4
    t0 = jnp.broadcast_to(t_ref[0:1, :], (R, C))
    t1 = jnp.broadcast_to(t_ref[1:2, :], (R, C))
    r0 = jnp.broadcast_to(r_ref[0:1, :], (R, C))
    r1 = jnp.broadcast_to(r_ref[1:2, :], (R, C))
    o0 = jnp.broadcast_to(o_ref[0:1, :], (R, C))
    o1 = jnp.broadcast_to(o_ref[1:2, :], (R, C))
    tv = jnp.where(sm4 < 2, t0, t1)
    rv = jnp.where((sm4 == 0) | (sm4 == 3), r0, r1)
    ov = jnp.where(q == 0, 0.5 * (o0 + o1), jnp.where(q == 1, o0, o1))
    sarv = jnp.where(c < DIM1, tv, jnp.where(c < 2 * DIM1, rv, ov))
    out_ref[...] = jnp.where(r < NUM_SAR, sarv, sincos)


def _build_table(embed_transmit, embed_receive, embed_orbit):
    f32 = jnp.float32
    t = jnp.zeros((2, EMBED_DIM), f32).at[:, 0:DIM1].set(embed_transmit)
    r = jnp.zeros((2, EMBED_DIM), f32).at[:, DIM1:2 * DIM1].set(embed_receive)
    o = jnp.zeros((2, EMBED_DIM), f32).at[:, 2 * DIM1:].set(embed_orbit)
    return pl.pallas_call(
        _table_body,
        out_shape=jax.ShapeDtypeStruct((NUM_ROWS, EMBED_DIM), f32),
    )(t, r, o)


_GATHER_SPLITS = [(0, 128), (128, 72), (200, 128), (328, 72)]


def _make_gather(half):
  @functools.partial(
      pl.kernel,
      out_type=jax.ShapeDtypeStruct((NBH, SEQ, EMBED_DIM), jnp.float32),
      mesh=plsc.VectorSubcoreMesh(core_axis_name="c", subcore_axis_name="s"),
      scratch_types=[
          pltpu.VMEM((2 * ROWS,), jnp.int32),
          pltpu.VMEM((2, CB, SEQ, EMBED_DIM), jnp.float32),
          pltpu.VMEM_SHARED((NUM_ROWS, EMBED_DIM), jnp.float32),
          pltpu.SemaphoreType.DMA,
          pltpu.SemaphoreType.DMA,
          pltpu.SemaphoreType.DMA,
          pltpu.SemaphoreType.DMA,
          pltpu.SemaphoreType.DMA,
          pltpu.SemaphoreType.DMA,
      ],
      compiler_params=pltpu.CompilerParams(use_tc_tiling_on_sc=False),
  )
  def _gather(table_hbm, idx_hbm, out_hbm, idx_v, rows_v, table_sh,
              sem_i0, sem_i1, sem_g0, sem_g1, sem_o0, sem_o1):
    wid = lax.axis_index("s") * NC + lax.axis_index("c")
    rbase = (half * NBH + wid * BPW) * SEQ
    bbase = wid * BPW
    sem_i = (sem_i0, sem_i1)
    sem_g = (sem_g0, sem_g1)
    sem_o = (sem_o0, sem_o1)

    @pl.when(lax.axis_index("s") == 0)
    def _():
        pltpu.sync_copy(table_hbm, table_sh)
    plsc.subcore_barrier()

    pltpu.async_copy(idx_hbm.at[pl.ds(rbase, ROWS)],
                     idx_v.at[pl.ds(0, ROWS)], sem_i[0])

    def body(i, carry):
        for b in range(2):
            c = 2 * i + b
            roff = c * ROWS
            ioff = b * ROWS

            pltpu.make_async_copy(
                idx_hbm.at[pl.ds(rbase + roff, ROWS)],
                idx_v.at[pl.ds(ioff, ROWS)], sem_i[b]
            ).wait()

            @pl.when(c + 1 < NCHUNK)
            def _():
                pltpu.async_copy(
                    idx_hbm.at[pl.ds(rbase + roff + ROWS, ROWS)],
                    idx_v.at[pl.ds((1 - b) * ROWS, ROWS)],
                    sem_i[1 - b],
                )

            for k in range(ROWS // 16):
                sl = pl.ds(ioff + k * 16, 16)
                idx_v[sl] = idx_v[sl] + NUM_SAR

            @pl.when(i > 0)
            def _():
                pltpu.make_async_copy(
                    rows_v.at[b],
                    out_hbm.at[pl.ds(bbase + c * CB, CB)],
                    sem_o[b],
                ).wait()

            copies = [
                pltpu.async_copy(
                    table_sh.at[idx_v.at[pl.ds(ioff + start, n)]],
                    rows_v.at[b, start // SEQ, pl.ds(start % SEQ, n)],
                    sem_g[b],
                )
                for start, n in _GATHER_SPLITS
            ]
            for cp in copies:
                cp.wait()

            pltpu.async_copy(
                rows_v.at[b], out_hbm.at[pl.ds(bbase + c * CB, CB)], sem_o[b]
            )
        return carry

    lax.fori_loop(0, NCHUNK // 2, body, 0)

    pltpu.make_async_copy(
        rows_v.at[0], out_hbm.at[pl.ds(bbase + (NCHUNK - 2) * CB, CB)], sem_o[0]
    ).wait()
    pltpu.make_async_copy(
        rows_v.at[1], out_hbm.at[pl.ds(bbase + (NCHUNK - 1) * CB, CB)], sem_o[1]
    ).wait()

  return _gather


_gather_halves = [_make_gather(h) for h in range(NSPLIT)]


def _xpose_body(in_ref, out_ref):
    in3 = in_ref.reshape(TCB, SEQ // 2, 2 * EMBED_DIM)
    for t in range(SEQ // 2):
        xk = in3[:, t, :]
        out_ref[2 * t] = jnp.transpose(xk[:, 0:EMBED_DIM])
        out_ref[2 * t + 1] = jnp.transpose(xk[:, EMBED_DIM:])


def _xpose_body_acc(in_ref, prev_ref, out_ref):
    del prev_ref
    _xpose_body(in_ref, out_ref)


def _transpose_tc(out_sc, half, prev):
    t2 = out_sc.reshape(NBH * SEQ // 2, 2 * EMBED_DIM)
    rows_per_blk = TCB * SEQ // 2
    goff = half * (NBH // TCB)
    in_spec = pl.BlockSpec((rows_per_blk, 2 * EMBED_DIM), lambda g: (g, 0))
    out_spec = pl.BlockSpec((SEQ, EMBED_DIM, TCB), lambda g: (0, 0, g + goff))
    out_shape = jax.ShapeDtypeStruct((SEQ, EMBED_DIM, BATCH), jnp.float32)
    if prev is None:
        return pl.pallas_call(
            _xpose_body,
            grid=(NBH // TCB,),
            in_specs=[in_spec],
            out_specs=out_spec,
            out_shape=out_shape,
        )(t2)
    return pl.pallas_call(
        _xpose_body_acc,
        grid=(NBH // TCB,),
        in_specs=[in_spec,
                  pl.BlockSpec((8, 8, 128), lambda g: (0, 0, 0))],
        out_specs=out_spec,
        out_shape=out_shape,
        input_output_aliases={1: 0},
    )(t2, prev)


def kernel(input, embed_transmit, embed_receive, embed_orbit):
    table = _build_table(embed_transmit, embed_receive, embed_orbit)
    idx = input.reshape(-1).astype(jnp.int32)
    tr = None
    for h in range(NSPLIT):
        out_sc = _gather_halves[h](table, idx)
        tr = _transpose_tc(out_sc, h, tr)
    return jnp.transpose(tr, (2, 0, 1))


# speedup vs baseline: 3.1954x; 1.0071x over previous
import functools
import math

import jax
import jax.numpy as jnp
from jax import lax
from jax.experimental import pallas as pl
from jax.experimental.pallas import tpu as pltpu
from jax.experimental.pallas import tpu_sc as plsc

EMBED_DIM = 64
DIM1 = EMBED_DIM // 3
NUM_SAR = 12
NUM_OPT = 2500
NUM_ROWS = NUM_SAR + NUM_OPT

BATCH = 4096
SEQ = 200

NC, NS = 2, 16
NW = NC * NS
NSPLIT = 4
NBH = BATCH // NSPLIT
BPW = NBH // NW
CB = 2
ROWS = CB * SEQ
NCHUNK = BPW // CB
TCB = 128


def _table_body(t_ref, r_ref, o_ref, out_ref):
    R, C = NUM_ROWS, EMBED_DIM
    r = lax.broadcasted_iota(jnp.int32, (R, C), 0)
    c = lax.broadcasted_iota(jnp.int32, (R, C), 1)
    pos = (r - NUM_SAR).astype(jnp.float32)
    j = (c % 32).astype(jnp.float32)
    omega = jnp.exp(j * (-math.log(10000.0) / 32.0))
    ang = pos * omega
    sincos = jnp.where(c < 32, jnp.sin(ang), jnp.cos(ang))
    s = 11 - r
    sm4 = s % 4
    q = s // 4
    t0 = jnp.broadcast_to(t_ref[0:1, :], (R, C))
    t1 = jnp.broadcast_to(t_ref[1:2, :], (R, C))
    r0 = jnp.broadcast_to(r_ref[0:1, :], (R, C))
    r1 = jnp.broadcast_to(r_ref[1:2, :], (R, C))
    o0 = jnp.broadcast_to(o_ref[0:1, :], (R, C))
    o1 = jnp.broadcast_to(o_ref[1:2, :], (R, C))
    tv = jnp.where(sm4 < 2, t0, t1)
    rv = jnp.where((sm4 == 0) | (sm4 == 3), r0, r1)
    ov = jnp.where(q == 0, 0.5 * (o0 + o1), jnp.where(q == 1, o0, o1))
    sarv = jnp.where(c < DIM1, tv, jnp.where(c < 2 * DIM1, rv, ov))
    out_ref[...] = jnp.where(r < NUM_SAR, sarv, sincos)


def _build_table(embed_transmit, embed_receive, embed_orbit):
    f32 = jnp.float32
    t = jnp.zeros((2, EMBED_DIM), f32).at[:, 0:DIM1].set(embed_transmit)
    r = jnp.zeros((2, EMBED_DIM), f32).at[:, DIM1:2 * DIM1].set(embed_receive)
    o = jnp.zeros((2, EMBED_DIM), f32).at[:, 2 * DIM1:].set(embed_orbit)
    return pl.pallas_call(
        _table_body,
        out_shape=jax.ShapeDtypeStruct((NUM_ROWS, EMBED_DIM), f32),
    )(t, r, o)


_GATHER_SPLITS = [(0, 128), (128, 72), (200, 128), (328, 72)]


def _make_gather(half):
  @functools.partial(
      pl.kernel,
      out_type=jax.ShapeDtypeStruct((NBH, SEQ, EMBED_DIM), jnp.float32),
      mesh=plsc.VectorSubcoreMesh(core_axis_name="c", subcore_axis_name="s"),
      scratch_types=[
          pltpu.VMEM((2 * ROWS,), jnp.int32),
          pltpu.VMEM((2, CB, SEQ, EMBED_DIM), jnp.float32),
          pltpu.VMEM_SHARED((NUM_ROWS, EMBED_DIM), jnp.float32),
          pltpu.SemaphoreType.DMA,
          pltpu.SemaphoreType.DMA,
          pltpu.SemaphoreType.DMA,
          pltpu.SemaphoreType.DMA,
          pltpu.SemaphoreType.DMA,
          pltpu.SemaphoreType.DMA,
      ],
      compiler_params=pltpu.CompilerParams(use_tc_tiling_on_sc=False),
  )
  def _gather(table_hbm, idx_hbm, out_hbm, idx_v, rows_v, table_sh,
              sem_i0, sem_i1, sem_g0, sem_g1, sem_o0, sem_o1):
    wid = lax.axis_index("s") * NC + lax.axis_index("c")
    rbase = (half * NBH + wid * BPW) * SEQ
    bbase = wid * BPW
    sem_i = (sem_i0, sem_i1)
    sem_g = (sem_g0, sem_g1)
    sem_o = (sem_o0, sem_o1)

    @pl.when(lax.axis_index("s") == 0)
    def _():
        pltpu.sync_copy(table_hbm, table_sh)
    plsc.subcore_barrier()

    pltpu.async_copy(idx_hbm.at[pl.ds(rbase, ROWS)],
                     idx_v.at[pl.ds(0, ROWS)], sem_i[0])

    def body(i, carry):
        for b in range(2):
            c = 2 * i + b
            roff = c * ROWS
            ioff = b * ROWS

            pltpu.make_async_copy(
                idx_hbm.at[pl.ds(rbase + roff, ROWS)],
                idx_v.at[pl.ds(ioff, ROWS)], sem_i[b]
            ).wait()

            @pl.when(c + 1 < NCHUNK)
            def _():
                pltpu.async_copy(
                    idx_hbm.at[pl.ds(rbase + roff + ROWS, ROWS)],
                    idx_v.at[pl.ds((1 - b) * ROWS, ROWS)],
                    sem_i[1 - b],
                )

            for k in range(ROWS // 16):
                sl = pl.ds(ioff + k * 16, 16)
                idx_v[sl] = idx_v[sl] + NUM_SAR

            @pl.when(i > 0)
            def _():
                pltpu.make_async_copy(
                    rows_v.at[b],
                    out_hbm.at[pl.ds(bbase + c * CB, CB)],
                    sem_o[b],
                ).wait()

            copies = [
                pltpu.async_copy(
                    table_sh.at[idx_v.at[pl.ds(ioff + start, n)]],
                    rows_v.at[b, start // SEQ, pl.ds(start % SEQ, n)],
                    sem_g[b],
                )
                for start, n in _GATHER_SPLITS
            ]
            for cp in copies:
                cp.wait()

            pltpu.async_copy(
                rows_v.at[b], out_hbm.at[pl.ds(bbase + c * CB, CB)], sem_o[b]
            )
        return carry

    lax.fori_loop(0, NCHUNK // 2, body, 0)

    pltpu.make_async_copy(
        rows_v.at[0], out_hbm.at[pl.ds(bbase + (NCHUNK - 2) * CB, CB)], sem_o[0]
    ).wait()
    pltpu.make_async_copy(
        rows_v.at[1], out_hbm.at[pl.ds(bbase + (NCHUNK - 1) * CB, CB)], sem_o[1]
    ).wait()

  return _gather


_gather_halves = [_make_gather(h) for h in range(NSPLIT)]


def _xpose_body(in_ref, out_ref):
    in3 = in_ref.reshape(TCB, SEQ // 2, 2 * EMBED_DIM)
    for t in range(SEQ // 2):
        xk = in3[:, t, :]
        out_ref[2 * t] = jnp.transpose(xk[:, 0:EMBED_DIM])
        out_ref[2 * t + 1] = jnp.transpose(xk[:, EMBED_DIM:])


def _xpose_body_acc(in_ref, prev_ref, out_ref):
    del prev_ref
    _xpose_body(in_ref, out_ref)


def _transpose_tc(out_sc, half, prev):
    t2 = out_sc.reshape(NBH * SEQ // 2, 2 * EMBED_DIM)
    rows_per_blk = TCB * SEQ // 2
    goff = half * (NBH // TCB)
    in_spec = pl.BlockSpec((rows_per_blk, 2 * EMBED_DIM), lambda g: (g, 0))
    out_spec = pl.BlockSpec((SEQ, EMBED_DIM, TCB), lambda g: (0, 0, g + goff))
    out_shape = jax.ShapeDtypeStruct((SEQ, EMBED_DIM, BATCH), jnp.float32)
    if prev is None:
        return pl.pallas_call(
            _xpose_body,
            grid=(NBH // TCB,),
            in_specs=[in_spec],
            out_specs=out_spec,
            out_shape=out_shape,
        )(t2)
    return pl.pallas_call(
        _xpose_body_acc,
        grid=(NBH // TCB,),
        in_specs=[in_spec,
                  pl.BlockSpec((8, 8, 128), lambda g: (0, 0, 0))],
        out_specs=out_spec,
        out_shape=out_shape,
        input_output_aliases={1: 0},
    )(t2, prev)


def kernel(input, embed_transmit, embed_receive, embed_orbit):
    table = _build_table(embed_transmit, embed_receive, embed_orbit)
    idx = input.reshape(-1).astype(jnp.int32)
    tr = None
    for h in range(NSPLIT):
        out_sc = _gather_halves[h](table, idx)
        tr = _transpose_tc(out_sc, h, tr)
    return jnp.transpose(tr, (2, 0, 1))
